# Initial kernel scaffold; baseline (speedup 1.0000x reference)
import jax
import jax.numpy as jnp
from jax.experimental import pallas as pl


def kernel(x, b, hout, wout, c1_idxes, c2_idxes, shift_pads):
    raise NotImplementedError("write your pallas kernel here")


# speedup vs baseline: 3.4090x; 3.4090x over previous
import jax
import jax.numpy as jnp
from jax import lax
from jax.experimental import pallas as pl
from jax.experimental.pallas import tpu as pltpu
from jax.experimental.pallas import tpu_sc as plsc

_PAD_LK = 25
_EXTRA = 23
_C_OUT = 192
_L = 16


def kernel(x, b, hout, wout, c1_idxes, c2_idxes, shift_pads):
    del b, hout, wout
    B, C_IN, HIN, WIN = x.shape
    HOUT, WOUT = HIN - _PAD_LK, WIN - _PAD_LK
    KW = WOUT // _L

    info = plsc.get_sparse_core_info()
    NC, NS = info.num_cores, info.num_subcores
    NW = NC * NS
    JPW = _C_OUT // NW

    mesh = plsc.VectorSubcoreMesh(core_axis_name="c", subcore_axis_name="s")
    out_sds = jax.ShapeDtypeStruct((B, _C_OUT, HOUT, WOUT), jnp.float32)

    def body(x_hbm, c1_hbm, c2_hbm, sp_hbm, out1_hbm, out2_hbm,
             idx_v, c1_sm, c2_sm, sp_sm, mc_sm, ms_sm, buf, acc):
        wid = lax.axis_index("s") * NC + lax.axis_index("c")

        lane = lax.iota(jnp.int32, _L)
        for hbm_ref, sm_ref in ((c1_hbm, c1_sm), (c2_hbm, c2_sm), (sp_hbm, sp_sm)):
            pltpu.sync_copy(hbm_ref, idx_v)

            def stage(i, carry, sm_ref=sm_ref):
                v = idx_v[pl.ds((i // _L) * _L, _L)]
                sm_ref[i] = jnp.max(jnp.where(lane == i % _L, v, jnp.int32(-2147483647)))
                return carry

            lax.fori_loop(0, C_IN, stage, 0)

        zero_v = jnp.zeros((_L,), jnp.float32)

        def run_items(idx_sm, out_hbm, bi, horizontal):
            def item(jj, _):
                j = wid * JPW + jj

                def scan(c, n):
                    hit = idx_sm[c] == j

                    @pl.when(hit)
                    def _():
                        mc_sm[n] = c
                        ms_sm[n] = sp_sm[c]

                    return jnp.where(hit, n + 1, n)

                n = lax.fori_loop(0, C_IN, scan, jnp.int32(0))

                def start_load(m):
                    c = mc_sm[m]
                    s = ms_sm[m]
                    h0 = _EXTRA if horizontal else s
                    pltpu.sync_copy(x_hbm.at[bi, c, pl.ds(h0, HOUT), :], buf)
                    return s if horizontal else _EXTRA

                @pl.when(n == 0)
                def _():
                    def zrow(h, carry):
                        for k in range(KW):
                            acc[h, pl.ds(k * _L, _L)] = zero_v
                        return carry

                    lax.fori_loop(0, HOUT, zrow, 0)

                @pl.when(n > 0)
                def _():
                    woff0 = start_load(jnp.int32(0))

                    def crow(h, woff):
                        for k in range(KW):
                            acc[h, pl.ds(k * _L, _L)] = buf[h, pl.ds(woff + k * _L, _L)]
                        return woff

                    lax.fori_loop(0, HOUT, crow, woff0)

                    def more(m, carry):
                        woff = start_load(m)

                        def arow(h, woff):
                            for k in range(KW):
                                plsc.addupdate(acc.at[h, pl.ds(k * _L, _L)],
                                               buf[h, pl.ds(woff + k * _L, _L)])
                            return woff

                        lax.fori_loop(0, HOUT, arow, woff)
                        return carry

                    lax.fori_loop(1, n, more, 0)

                pltpu.sync_copy(acc, out_hbm.at[bi, j])
                return 0

            lax.fori_loop(0, JPW, item, 0)

        for bi in range(B):
            run_items(c1_sm, out1_hbm, bi, True)
            run_items(c2_sm, out2_hbm, bi, False)

    kfn = pl.kernel(
        body,
        out_type=(out_sds, out_sds),
        mesh=mesh,
        compiler_params=pltpu.CompilerParams(use_tc_tiling_on_sc=False, needs_layout_passes=False),
        scratch_types=[
            pltpu.VMEM((C_IN,), jnp.int32),
            pltpu.SMEM((C_IN,), jnp.int32),
            pltpu.SMEM((C_IN,), jnp.int32),
            pltpu.SMEM((C_IN,), jnp.int32),
            pltpu.SMEM((C_IN,), jnp.int32),
            pltpu.SMEM((C_IN,), jnp.int32),
            pltpu.VMEM((HOUT, WIN), jnp.float32),
            pltpu.VMEM((HOUT, WOUT), jnp.float32),
        ],
    )
    return kfn(x, c1_idxes.astype(jnp.int32), c2_idxes.astype(jnp.int32),
               shift_pads.astype(jnp.int32))


# speedup vs baseline: 4.9662x; 1.4568x over previous
import jax
import jax.numpy as jnp
from jax import lax
from jax.experimental import pallas as pl
from jax.experimental.pallas import tpu as pltpu
from jax.experimental.pallas import tpu_sc as plsc

_PAD_LK = 25
_EXTRA = 23
_C_OUT = 192
_L = 16


def kernel(x, b, hout, wout, c1_idxes, c2_idxes, shift_pads):
    del b, hout, wout
    B, C_IN, HIN, WIN = x.shape
    HOUT, WOUT = HIN - _PAD_LK, WIN - _PAD_LK
    KW = WOUT // _L

    info = plsc.get_sparse_core_info()
    NC, NS = info.num_cores, info.num_subcores
    NW = NC * NS
    JPW = _C_OUT // NW

    mesh = plsc.VectorSubcoreMesh(core_axis_name="c", subcore_axis_name="s")
    out_sds = jax.ShapeDtypeStruct((B, _C_OUT, HOUT, WOUT), jnp.float32)

    def body(x_hbm, c1_hbm, c2_hbm, sp_hbm, out1_hbm, out2_hbm,
             idx_v, c1_sm, c2_sm, sp_sm, mc_sm, ms_sm, buf, acc):
        wid = lax.axis_index("s") * NC + lax.axis_index("c")

        lane = lax.iota(jnp.int32, _L)
        for hbm_ref, sm_ref in ((c1_hbm, c1_sm), (c2_hbm, c2_sm), (sp_hbm, sp_sm)):
            pltpu.sync_copy(hbm_ref, idx_v)

            def stage(i, carry, sm_ref=sm_ref):
                v = idx_v[pl.ds((i // _L) * _L, _L)]
                sm_ref[i] = jnp.max(jnp.where(lane == i % _L, v, jnp.int32(-2147483647)))
                return carry

            lax.fori_loop(0, C_IN, stage, 0)

        zero_v = jnp.zeros((_L,), jnp.float32)

        def run_items(idx_sm, out_hbm, bi, horizontal):
            def item(jj, _):
                j = wid * JPW + jj

                def scan(c, n):
                    hit = idx_sm[c] == j

                    @pl.when(hit)
                    def _():
                        mc_sm[n] = c
                        ms_sm[n] = sp_sm[c]

                    return jnp.where(hit, n + 1, n)

                n = lax.fori_loop(0, C_IN, scan, jnp.int32(0))

                def col_off(m):
                    return ms_sm[m] if horizontal else jnp.int32(_EXTRA)

                def row_off(m):
                    return jnp.int32(_EXTRA) if horizontal else ms_sm[m]

                @pl.when(n == 0)
                def _():
                    @plsc.parallel_loop(0, HOUT, unroll=4)
                    def zrow(h):
                        for k in range(KW):
                            acc[h, pl.ds(k * _L, _L)] = zero_v

                @pl.when(n > 0)
                def _():
                    pltpu.sync_copy(
                        x_hbm.at[bi, mc_sm[0], pl.ds(row_off(jnp.int32(0)), HOUT), :],
                        buf)
                    w0 = col_off(jnp.int32(0))

                    @plsc.parallel_loop(0, HOUT, unroll=4)
                    def crow(h):
                        for k in range(KW):
                            acc[h, pl.ds(k * _L, _L)] = buf[h, pl.ds(w0 + k * _L, _L)]

                    def more(m, carry):
                        pltpu.sync_copy(
                            x_hbm.at[bi, mc_sm[m], pl.ds(row_off(m), HOUT), :], buf)
                        sm = col_off(m)

                        @plsc.parallel_loop(0, HOUT, unroll=4)
                        def arow(h):
                            for k in range(KW):
                                plsc.addupdate(acc.at[h, pl.ds(k * _L, _L)],
                                               buf[h, pl.ds(sm + k * _L, _L)])
                        return carry

                    lax.fori_loop(1, n, more, 0)

                pltpu.sync_copy(acc, out_hbm.at[bi, j])
                return 0

            lax.fori_loop(0, JPW, item, 0)

        for bi in range(B):
            run_items(c1_sm, out1_hbm, bi, True)
            run_items(c2_sm, out2_hbm, bi, False)

    kfn = pl.kernel(
        body,
        out_type=(out_sds, out_sds),
        mesh=mesh,
        compiler_params=pltpu.CompilerParams(use_tc_tiling_on_sc=False, needs_layout_passes=False),
        scratch_types=[
            pltpu.VMEM((C_IN,), jnp.int32),
            pltpu.SMEM((C_IN,), jnp.int32),
            pltpu.SMEM((C_IN,), jnp.int32),
            pltpu.SMEM((C_IN,), jnp.int32),
            pltpu.SMEM((C_IN,), jnp.int32),
            pltpu.SMEM((C_IN,), jnp.int32),
            pltpu.VMEM((HOUT, WIN), jnp.float32),
            pltpu.VMEM((HOUT, WOUT), jnp.float32),
        ],
    )
    return kfn(x, c1_idxes.astype(jnp.int32), c2_idxes.astype(jnp.int32),
               shift_pads.astype(jnp.int32))


# speedup vs baseline: 5.3023x; 1.0677x over previous
import jax
import jax.numpy as jnp
from jax import lax
from jax.experimental import pallas as pl
from jax.experimental.pallas import tpu as pltpu
from jax.experimental.pallas import tpu_sc as plsc

_PAD_LK = 25
_EXTRA = 23
_C_OUT = 192
_L = 16


def kernel(x, b, hout, wout, c1_idxes, c2_idxes, shift_pads):
    del b, hout, wout
    B, C_IN, HIN, WIN = x.shape
    HOUT, WOUT = HIN - _PAD_LK, WIN - _PAD_LK
    KW = WOUT // _L
    NK = C_IN // _L
    WB = WIN

    info = plsc.get_sparse_core_info()
    NC, NS = info.num_cores, info.num_subcores
    NW = NC * NS
    JPW = _C_OUT // NW

    mesh = plsc.VectorSubcoreMesh(core_axis_name="c", subcore_axis_name="s")
    out_sds = jax.ShapeDtypeStruct((B, _C_OUT, HOUT, WOUT), jnp.float32)

    def body(x_hbm, c1_hbm, c2_hbm, sp_hbm, out1_hbm, out2_hbm,
             c1_v, c2_v, c1_sm, c2_sm, sp_sm, mc_sm, ms_sm, buf, acc, dsem):
        wid = lax.axis_index("s") * NC + lax.axis_index("c")
        j0 = wid * JPW

        lane = lax.iota(jnp.int32, _L)

        def stage_smem(src_v, sm_ref):
            def stage(i, carry):
                v = src_v[pl.ds((i // _L) * _L, _L)]
                sm_ref[i] = jnp.max(jnp.where(lane == i % _L, v, jnp.int32(-2147483647)))
                return carry

            lax.fori_loop(0, C_IN, stage, 0)

        pltpu.sync_copy(sp_hbm, c1_v)
        stage_smem(c1_v, sp_sm)
        pltpu.sync_copy(c1_hbm, c1_v)
        stage_smem(c1_v, c1_sm)
        pltpu.sync_copy(c2_hbm, c2_v)
        stage_smem(c2_v, c2_sm)

        zero_v = jnp.zeros((_L,), jnp.float32)

        def src_band(m_c, m_s, bi, horizontal):
            r0 = jnp.int32(_EXTRA) if horizontal else m_s
            return x_hbm.at[bi, m_c, pl.ds(r0, HOUT), :]

        def rem_of(m_s, horizontal):
            return m_s if horizontal else jnp.int32(_EXTRA)

        def run_segment(cidx_v, c_sm, out_hbm, bi, horizontal):
            def scan_into(jscan, off):
                def chunk(k, n):
                    v = cidx_v[pl.ds(k * _L, _L)]

                    def sub(nn):
                        def lane_scan(l, nn2):
                            kl = k * _L + l
                            hl = c_sm[kl] == jscan

                            @pl.when(hl)
                            def _():
                                mc_sm[off + nn2] = kl
                                ms_sm[off + nn2] = sp_sm[kl]

                            return jnp.where(hl, nn2 + 1, nn2)

                        return lax.fori_loop(0, _L, lane_scan, nn)

                    return lax.cond(jnp.any(v == jscan), sub, lambda nn: nn, n)

                return lax.fori_loop(0, NK, chunk, jnp.int32(0))

            def issue_first(off):
                pltpu.async_copy(
                    src_band(mc_sm[off], ms_sm[off], bi, horizontal), buf, dsem)

            def wait_first():
                pltpu.make_async_copy(
                    x_hbm.at[0, 0, pl.ds(0, HOUT), pl.ds(0, WB)], buf, dsem).wait()

            n0 = scan_into(j0, 0)

            @pl.when(n0 > 0)
            def _():
                issue_first(0)

            def item(jj, n):
                j = j0 + jj
                off = (jj % 2) * C_IN
                offn = ((jj + 1) % 2) * C_IN

                @pl.when(n == 0)
                def _():
                    @plsc.parallel_loop(0, HOUT, unroll=4)
                    def zrow(h):
                        for k in range(KW):
                            acc[h, pl.ds(k * _L, _L)] = zero_v

                @pl.when(n > 0)
                def _():
                    wait_first()
                    rem = rem_of(ms_sm[off], horizontal)

                    @plsc.parallel_loop(0, HOUT, unroll=4)
                    def crow(h):
                        for k in range(KW):
                            acc[h, pl.ds(k * _L, _L)] = buf[h, pl.ds(rem + k * _L, _L)]

                    def more(m, carry):
                        pltpu.sync_copy(
                            src_band(mc_sm[off + m], ms_sm[off + m], bi, horizontal),
                            buf)
                        remm = rem_of(ms_sm[off + m], horizontal)

                        @plsc.parallel_loop(0, HOUT, unroll=4)
                        def arow(h):
                            for k in range(KW):
                                plsc.addupdate(acc.at[h, pl.ds(k * _L, _L)],
                                               buf[h, pl.ds(remm + k * _L, _L)])

                        return carry

                    lax.fori_loop(1, n, more, 0)

                jscan = jnp.where(jj < JPW - 1, j + 1, _C_OUT + wid)
                n_next = scan_into(jscan, offn)

                @pl.when(n_next > 0)
                def _():
                    issue_first(offn)

                pltpu.sync_copy(acc, out_hbm.at[bi, j])
                return n_next

            lax.fori_loop(0, JPW, item, n0)

        for bi in range(B):
            run_segment(c1_v, c1_sm, out1_hbm, bi, True)
            run_segment(c2_v, c2_sm, out2_hbm, bi, False)

    kfn = pl.kernel(
        body,
        out_type=(out_sds, out_sds),
        mesh=mesh,
        compiler_params=pltpu.CompilerParams(use_tc_tiling_on_sc=False,
                                             needs_layout_passes=False),
        scratch_types=[
            pltpu.VMEM((C_IN,), jnp.int32),
            pltpu.VMEM((C_IN,), jnp.int32),
            pltpu.SMEM((C_IN,), jnp.int32),
            pltpu.SMEM((C_IN,), jnp.int32),
            pltpu.SMEM((C_IN,), jnp.int32),
            pltpu.SMEM((2 * C_IN,), jnp.int32),
            pltpu.SMEM((2 * C_IN,), jnp.int32),
            pltpu.VMEM((HOUT, WB), jnp.float32),
            pltpu.VMEM((HOUT, WOUT), jnp.float32),
            pltpu.SemaphoreType.DMA,
        ],
    )
    return kfn(x, c1_idxes.astype(jnp.int32), c2_idxes.astype(jnp.int32),
               shift_pads.astype(jnp.int32))


# speedup vs baseline: 5.4511x; 1.0281x over previous
import jax
import jax.numpy as jnp
from jax import lax
from jax.experimental import pallas as pl
from jax.experimental.pallas import tpu as pltpu
from jax.experimental.pallas import tpu_sc as plsc

_PAD_LK = 25
_EXTRA = 23
_C_OUT = 192
_L = 16


def kernel(x, b, hout, wout, c1_idxes, c2_idxes, shift_pads):
    del b, hout, wout
    B, C_IN, HIN, WIN = x.shape
    HOUT, WOUT = HIN - _PAD_LK, WIN - _PAD_LK
    KW = WOUT // _L
    NK = C_IN // _L
    CH = HOUT // 2

    info = plsc.get_sparse_core_info()
    NC, NS = info.num_cores, info.num_subcores
    NW = NC * NS
    JPW = _C_OUT // NW

    mesh = plsc.VectorSubcoreMesh(core_axis_name="c", subcore_axis_name="s")
    out_sds = jax.ShapeDtypeStruct((B, _C_OUT, HOUT, WOUT), jnp.float32)

    def body(x_hbm, c1_hbm, c2_hbm, sp_hbm, out1_hbm, out2_hbm,
             idx_v, c1_sm, c2_sm, sp_sm, mc_sm, ms_sm,
             buf0, buf1, acc0, acc1, sem_in, sem_out):
        wid = lax.axis_index("s") * NC + lax.axis_index("c")
        j0 = wid * JPW

        lane = lax.iota(jnp.int32, _L)

        def stage_smem(sm_ref):
            def stage(i, carry):
                v = idx_v[pl.ds((i // _L) * _L, _L)]
                sm_ref[i] = jnp.max(jnp.where(lane == i % _L, v, jnp.int32(-2147483647)))
                return carry

            lax.fori_loop(0, C_IN, stage, 0)

        pltpu.sync_copy(sp_hbm, idx_v)
        stage_smem(sp_sm)
        pltpu.sync_copy(c1_hbm, idx_v)
        stage_smem(c1_sm)
        pltpu.sync_copy(c2_hbm, idx_v)
        stage_smem(c2_sm)

        zero_v = jnp.zeros((_L,), jnp.float32)

        def run_segment(cidx_hbm, c_sm, out_hbm, bi, horizontal):
            pltpu.sync_copy(cidx_hbm, idx_v)

            def src_band(m_c, m_s, half):
                r0 = jnp.int32(_EXTRA) if horizontal else m_s
                return x_hbm.at[bi, m_c, pl.ds(r0 + half * CH, CH), :]

            def rem_of(m_s):
                return m_s if horizontal else jnp.int32(_EXTRA)

            def scan_into(jscan, off):
                def chunk(k, n):
                    v = idx_v[pl.ds(k * _L, _L)]

                    def sub(nn):
                        def lane_scan(l, nn2):
                            kl = k * _L + l
                            hl = c_sm[kl] == jscan

                            @pl.when(hl)
                            def _():
                                mc_sm[off + nn2] = kl
                                ms_sm[off + nn2] = sp_sm[kl]

                            return jnp.where(hl, nn2 + 1, nn2)

                        return lax.fori_loop(0, _L, lane_scan, nn)

                    return lax.cond(jnp.any(v == jscan), sub, lambda nn: nn, n)

                return lax.fori_loop(0, NK, chunk, jnp.int32(0))

            def issue_in(off, half, buf_h):
                pltpu.async_copy(src_band(mc_sm[off], ms_sm[off], half),
                                 buf_h, sem_in)

            def wait_in(buf_h):
                pltpu.make_async_copy(
                    x_hbm.at[0, 0, pl.ds(0, CH), :], buf_h, sem_in).wait()

            def wait_out(acc_h):
                pltpu.make_async_copy(
                    acc_h, out_hbm.at[0, 0, pl.ds(0, CH), :], sem_out).wait()

            n0 = scan_into(j0, 0)

            @pl.when(n0 > 0)
            def _():
                issue_in(0, 0, buf0)

            def item(jj, n):
                j = j0 + jj
                off = (jj % 2) * C_IN
                offn = ((jj + 1) % 2) * C_IN
                n_next = n

                for half in (0, 1):
                    buf_h, acc_h = (buf0, acc0) if half == 0 else (buf1, acc1)

                    @pl.when(n > 0)
                    def _(buf_h=buf_h):
                        wait_in(buf_h)

                    if half == 0:
                        @pl.when(n > 0)
                        def _():
                            issue_in(off, 1, buf1)
                    else:
                        jscan = jnp.where(jj < JPW - 1, j + 1, _C_OUT + wid)
                        n_next = scan_into(jscan, offn)

                        @pl.when(n_next > 0)
                        def _():
                            issue_in(offn, 0, buf0)

                    @pl.when(jj >= 1)
                    def _(acc_h=acc_h):
                        wait_out(acc_h)

                    @pl.when(n == 0)
                    def _(acc_h=acc_h):
                        @plsc.parallel_loop(0, CH, unroll=4)
                        def zrow(h):
                            for k in range(KW):
                                acc_h[h, pl.ds(k * _L, _L)] = zero_v

                    @pl.when(n > 0)
                    def _(buf_h=buf_h, acc_h=acc_h, half=half):
                        rem = rem_of(ms_sm[off])

                        @plsc.parallel_loop(0, CH, unroll=4)
                        def crow(h):
                            for k in range(KW):
                                acc_h[h, pl.ds(k * _L, _L)] = \
                                    buf_h[h, pl.ds(rem + k * _L, _L)]

                        def more(m, carry):
                            pltpu.sync_copy(
                                src_band(mc_sm[off + m], ms_sm[off + m], half),
                                buf_h)
                            remm = rem_of(ms_sm[off + m])

                            @plsc.parallel_loop(0, CH, unroll=4)
                            def arow(h):
                                for k in range(KW):
                                    plsc.addupdate(
                                        acc_h.at[h, pl.ds(k * _L, _L)],
                                        buf_h[h, pl.ds(remm + k * _L, _L)])

                            return carry

                        lax.fori_loop(1, n, more, 0)

                    pltpu.async_copy(
                        acc_h, out_hbm.at[bi, j, pl.ds(half * CH, CH), :],
                        sem_out)

                return n_next

            lax.fori_loop(0, JPW, item, n0)
            wait_out(acc0)
            wait_out(acc1)

        for bi in range(B):
            run_segment(c1_hbm, c1_sm, out1_hbm, bi, True)
            run_segment(c2_hbm, c2_sm, out2_hbm, bi, False)

    kfn = pl.kernel(
        body,
        out_type=(out_sds, out_sds),
        mesh=mesh,
        compiler_params=pltpu.CompilerParams(use_tc_tiling_on_sc=False,
                                             needs_layout_passes=False),
        scratch_types=[
            pltpu.VMEM((C_IN,), jnp.int32),
            pltpu.SMEM((C_IN,), jnp.int32),
            pltpu.SMEM((C_IN,), jnp.int32),
            pltpu.SMEM((C_IN,), jnp.int32),
            pltpu.SMEM((2 * C_IN,), jnp.int32),
            pltpu.SMEM((2 * C_IN,), jnp.int32),
            pltpu.VMEM((CH, WIN), jnp.float32),
            pltpu.VMEM((CH, WIN), jnp.float32),
            pltpu.VMEM((CH, WOUT), jnp.float32),
            pltpu.VMEM((CH, WOUT), jnp.float32),
            pltpu.SemaphoreType.DMA,
            pltpu.SemaphoreType.DMA,
        ],
    )
    return kfn(x, c1_idxes.astype(jnp.int32), c2_idxes.astype(jnp.int32),
               shift_pads.astype(jnp.int32))
